# baseline (speedup 1.0000x reference)
import jax
import jax.numpy as jnp
from jax.experimental import pallas as pl
from jax.experimental.pallas import tpu as pltpu

_NNZ = 2684354
_KEEP = 0.7
_BLOCK = 524288
_SUB = 8


def _rotl(x, r):
    return (x << jnp.uint32(r)) | (x >> jnp.uint32(32 - r))


_ROT_A = (13, 15, 26, 6)
_ROT_B = (17, 29, 16, 24)


def _threefry_bits(i):
    ks0 = jnp.uint32(0)
    ks1 = jnp.uint32(42)
    ks2 = jnp.uint32(0x1BD11BDA ^ 42)
    ks = (ks0, ks1, ks2)
    x0 = jnp.full(i.shape, ks0, jnp.uint32)
    x1 = i + ks1
    for g in range(5):
        rots = _ROT_A if g % 2 == 0 else _ROT_B
        for r in rots:
            x0 = x0 + x1
            x1 = _rotl(x1, r)
            x1 = x1 ^ x0
        x0 = x0 + ks[(g + 1) % 3]
        x1 = x1 + ks[(g + 2) % 3] + jnp.uint32(g + 1)
    return x0 ^ x1


def _dropout_body(v_ref, o_ref):
    pid = pl.program_id(0)
    b = o_ref.shape[-1]
    rows, cols = _SUB, b // _SUB
    base = (pid * b).astype(jnp.uint32)
    flat = (jax.lax.broadcasted_iota(jnp.int32, (rows, cols), 0) * cols
            + jax.lax.broadcasted_iota(jnp.int32, (rows, cols), 1))
    i = base + flat.astype(jnp.uint32)
    bits = _threefry_bits(i)
    fbits = (bits >> jnp.uint32(9)) | jnp.uint32(0x3F800000)
    u = jax.lax.bitcast_convert_type(fbits, jnp.float32) - jnp.float32(1.0)
    scale = jnp.where(u < jnp.float32(_KEEP),
                      jnp.float32(1.0 / _KEEP), jnp.float32(0.0))
    o_ref[...] = v_ref[...] * scale.reshape(1, b)


def kernel(x_indices, x_values):
    del x_indices
    v2 = x_values.reshape(1, _NNZ)
    grid = pl.cdiv(_NNZ, _BLOCK)
    out = pl.pallas_call(
        _dropout_body,
        grid=(grid,),
        in_specs=[pl.BlockSpec((1, _BLOCK), lambda b: (0, b))],
        out_specs=pl.BlockSpec((1, _BLOCK), lambda b: (0, b)),
        out_shape=jax.ShapeDtypeStruct((1, _NNZ), jnp.float32),
        compiler_params=pltpu.CompilerParams(
            dimension_semantics=("parallel",),
        ),
    )(v2)
    return out.reshape(_NNZ)


# speedup vs baseline: 3.9375x; 3.9375x over previous
import jax
import jax.numpy as jnp
from jax.experimental import pallas as pl
from jax.experimental.pallas import tpu as pltpu

_NNZ = 2684354
_KEEP = 0.7
_BLOCK = 524288
_SUB = 8


def _rotl(x, r):
    return (x << jnp.uint32(r)) | (x >> jnp.uint32(32 - r))


_ROT_A = (13, 15, 26, 6)
_ROT_B = (17, 29, 16, 24)


def _threefry_bits(i):
    ks0 = jnp.uint32(0)
    ks1 = jnp.uint32(42)
    ks2 = jnp.uint32(0x1BD11BDA ^ 42)
    ks = (ks0, ks1, ks2)
    x0 = jnp.full(i.shape, ks0, jnp.uint32)
    x1 = i + ks1
    for g in range(5):
        rots = _ROT_A if g % 2 == 0 else _ROT_B
        for r in rots:
            x0 = x0 + x1
            x1 = _rotl(x1, r)
            x1 = x1 ^ x0
        x0 = x0 + ks[(g + 1) % 3]
        x1 = x1 + ks[(g + 2) % 3] + jnp.uint32(g + 1)
    return x0 ^ x1


_KEEP_BITS_THRESH = 0xB3333400


def _dropout_body(v_ref, o_ref):
    pid = pl.program_id(0)
    b = o_ref.shape[-1]
    rows, cols = _SUB, b // _SUB
    base = (pid * b).astype(jnp.uint32)
    flat = (jax.lax.broadcasted_iota(jnp.int32, (rows, cols), 0) * cols
            + jax.lax.broadcasted_iota(jnp.int32, (rows, cols), 1))
    i = base + flat.astype(jnp.uint32)
    bits = _threefry_bits(i)
    scale = jnp.where(bits < jnp.uint32(_KEEP_BITS_THRESH),
                      jnp.float32(1.0 / _KEEP), jnp.float32(0.0))
    o_ref[...] = (v_ref[...].reshape(rows, cols) * scale).reshape(b)


def kernel(x_indices, x_values):
    del x_indices
    grid = pl.cdiv(_NNZ, _BLOCK)
    out = pl.pallas_call(
        _dropout_body,
        grid=(grid,),
        in_specs=[pl.BlockSpec((_BLOCK,), lambda b: (b,))],
        out_specs=pl.BlockSpec((_BLOCK,), lambda b: (b,)),
        out_shape=jax.ShapeDtypeStruct((_NNZ,), jnp.float32),
        compiler_params=pltpu.CompilerParams(
            dimension_semantics=("parallel",),
        ),
    )(x_values)
    return out


# speedup vs baseline: 4.5065x; 1.1445x over previous
import jax
import jax.numpy as jnp
from jax.experimental import pallas as pl
from jax.experimental.pallas import tpu as pltpu

_NNZ = 2684354
_KEEP = 0.7
_BLOCK = 131072
_SUB = 8


def _rotl(x, r):
    return (x << jnp.uint32(r)) | (x >> jnp.uint32(32 - r))


_ROT_A = (13, 15, 26, 6)
_ROT_B = (17, 29, 16, 24)


def _threefry_bits(i):
    ks0 = jnp.uint32(0)
    ks1 = jnp.uint32(42)
    ks2 = jnp.uint32(0x1BD11BDA ^ 42)
    ks = (ks0, ks1, ks2)
    x0 = jnp.full(i.shape, ks0, jnp.uint32)
    x1 = i + ks1
    for g in range(5):
        rots = _ROT_A if g % 2 == 0 else _ROT_B
        for r in rots:
            x0 = x0 + x1
            x1 = _rotl(x1, r)
            x1 = x1 ^ x0
        x0 = x0 + ks[(g + 1) % 3]
        x1 = x1 + ks[(g + 2) % 3] + jnp.uint32(g + 1)
    return x0 ^ x1


_KEEP_BITS_THRESH = 0xB3333400


def _dropout_body(v_ref, o_ref):
    pid = pl.program_id(0)
    b = o_ref.shape[-1]
    rows, cols = _SUB, b // _SUB
    base = (pid * b).astype(jnp.uint32)
    flat = (jax.lax.broadcasted_iota(jnp.int32, (rows, cols), 0) * cols
            + jax.lax.broadcasted_iota(jnp.int32, (rows, cols), 1))
    i = base + flat.astype(jnp.uint32)
    bits = _threefry_bits(i)
    scale = jnp.where(bits < jnp.uint32(_KEEP_BITS_THRESH),
                      jnp.float32(1.0 / _KEEP), jnp.float32(0.0))
    o_ref[...] = (v_ref[...].reshape(rows, cols) * scale).reshape(b)


def kernel(x_indices, x_values):
    del x_indices
    grid = pl.cdiv(_NNZ, _BLOCK)
    out = pl.pallas_call(
        _dropout_body,
        grid=(grid,),
        in_specs=[pl.BlockSpec((_BLOCK,), lambda b: (b,))],
        out_specs=pl.BlockSpec((_BLOCK,), lambda b: (b,)),
        out_shape=jax.ShapeDtypeStruct((_NNZ,), jnp.float32),
        compiler_params=pltpu.CompilerParams(
            dimension_semantics=("parallel",),
        ),
    )(x_values)
    return out


# speedup vs baseline: 4.5405x; 1.0075x over previous
import jax
import jax.numpy as jnp
from jax.experimental import pallas as pl
from jax.experimental.pallas import tpu as pltpu

_NNZ = 2684354
_KEEP = 0.7
_BLOCK = 65536
_SUB = 8


def _rotl(x, r):
    return (x << jnp.uint32(r)) | (x >> jnp.uint32(32 - r))


_ROT_A = (13, 15, 26, 6)
_ROT_B = (17, 29, 16, 24)


def _threefry_bits(i):
    ks0 = jnp.uint32(0)
    ks1 = jnp.uint32(42)
    ks2 = jnp.uint32(0x1BD11BDA ^ 42)
    ks = (ks0, ks1, ks2)
    x0 = jnp.full(i.shape, ks0, jnp.uint32)
    x1 = i + ks1
    for g in range(5):
        rots = _ROT_A if g % 2 == 0 else _ROT_B
        for r in rots:
            x0 = x0 + x1
            x1 = _rotl(x1, r)
            x1 = x1 ^ x0
        x0 = x0 + ks[(g + 1) % 3]
        x1 = x1 + ks[(g + 2) % 3] + jnp.uint32(g + 1)
    return x0 ^ x1


_KEEP_BITS_THRESH = 0xB3333400


def _dropout_body(v_ref, o_ref):
    pid = pl.program_id(0)
    b = o_ref.shape[-1]
    rows, cols = _SUB, b // _SUB
    base = (pid * b).astype(jnp.uint32)
    flat = (jax.lax.broadcasted_iota(jnp.int32, (rows, cols), 0) * cols
            + jax.lax.broadcasted_iota(jnp.int32, (rows, cols), 1))
    i = base + flat.astype(jnp.uint32)
    bits = _threefry_bits(i)
    scale = jnp.where(bits < jnp.uint32(_KEEP_BITS_THRESH),
                      jnp.float32(1.0 / _KEEP), jnp.float32(0.0))
    o_ref[...] = (v_ref[...].reshape(rows, cols) * scale).reshape(b)


def kernel(x_indices, x_values):
    del x_indices
    grid = pl.cdiv(_NNZ, _BLOCK)
    out = pl.pallas_call(
        _dropout_body,
        grid=(grid,),
        in_specs=[pl.BlockSpec((_BLOCK,), lambda b: (b,))],
        out_specs=pl.BlockSpec((_BLOCK,), lambda b: (b,)),
        out_shape=jax.ShapeDtypeStruct((_NNZ,), jnp.float32),
        compiler_params=pltpu.CompilerParams(
            dimension_semantics=("parallel",),
        ),
    )(x_values)
    return out
